# baseline (speedup 1.0000x reference)
import functools

import jax
import jax.numpy as jnp
from jax import lax
from jax.experimental import pallas as pl
from jax.experimental.pallas import tpu as pltpu
from jax.experimental.pallas import tpu_sc as plsc

N = 4096
E = 131072
D_IN = 512
D_OUT = 256
ALPHA = 0.1
TEMP = 0.67

NC = 2
NS = 16
NW = NC * NS
EPT = E // NW
CHUNK = 128
RPT = EPT // CHUNK
NB = 128


def _threefry_xor(seed_lo, lo):
    ks0 = jnp.uint32(0)
    ks1 = jnp.uint32(seed_lo)
    ks2 = jnp.uint32(0x1BD11BDA) ^ ks1
    x0 = jnp.zeros_like(lo)
    x1 = lo + ks1
    rots = ((13, 15, 26, 6), (17, 29, 16, 24))
    sched = ((ks1, ks2, 1), (ks2, ks0, 2), (ks0, ks1, 3),
             (ks1, ks2, 4), (ks2, ks0, 5))
    for g in range(5):
        for r in rots[g % 2]:
            x0 = x0 + x1
            x1 = (x1 << r) | (x1 >> (32 - r))
            x1 = x1 ^ x0
        a, b, c = sched[g]
        x0 = x0 + a
        x1 = x1 + b + jnp.uint32(c)
    return x0 ^ x1


def _bits_to_unif(bits):
    f = lax.bitcast_convert_type((bits >> 9) | jnp.uint32(0x3F800000),
                                 jnp.float32)
    return jnp.maximum(f - 1.0, 0.0)


_LN2 = 0.6931471805599453


def _slog(v):
    bits = lax.bitcast_convert_type(v, jnp.int32)
    e = (bits >> 23) - 127
    m = lax.bitcast_convert_type((bits & 0x007FFFFF) | 0x3F800000,
                                 jnp.float32)
    s = (m - 1.0) / (m + 1.0)
    s2 = s * s
    p = jnp.full_like(v, 1.0 / 11.0)
    for c in (1.0 / 9.0, 1.0 / 7.0, 1.0 / 5.0, 1.0 / 3.0, 1.0):
        p = p * s2 + jnp.float32(c)
    return e.astype(jnp.float32) * jnp.float32(_LN2) + (s + s) * p


def _k1_body(rows_hbm, cols_hbm, lg_hbm, wt_hbm,
             ewm_hbm, degp_hbm, loopdp_hbm,
             rows_v, cols_v, z_v, t_v, w_v, ewm_v, ridx_c, dval_c, lval_c,
             zero_v, lg_v, deg_sp, loopd_sp, sem):
    c = lax.axis_index("c")
    s = lax.axis_index("s")
    wid = s * NC + c
    base = wid * RPT

    pltpu.sync_copy(rows_hbm.at[pl.ds(base, RPT)], rows_v)
    pltpu.sync_copy(cols_hbm.at[pl.ds(base, RPT)], cols_v)
    pltpu.sync_copy(lg_hbm, lg_v)
    lg = lg_v[...]

    @pl.when(s == 0)
    def _():
        z16 = jnp.zeros((16,), jnp.float32)

        def zb(i, carry):
            zero_v[pl.ds(i * 16, 16)] = z16
            return carry

        lax.fori_loop(0, N // 16, zb, 0)
        pltpu.sync_copy(zero_v, deg_sp)
        pltpu.sync_copy(zero_v, loopd_sp)

    plsc.subcore_barrier()

    def chunk_a(j, carry):
        for g in range(8):
            q = g * 16
            r = rows_v[j, pl.ds(q, 16)]
            cc = cols_v[j, pl.ds(q, 16)]
            k = lax.bitcast_convert_type((r << 12) + cc, jnp.uint32)
            u2 = jnp.clip(_bits_to_unif(_threefry_xor(43, k)),
                          1e-6, 1.0 - 1e-6)
            cst = _slog(u2 / (1.0 - u2))
            t = (lg + cst) * jnp.float32(1.0 / TEMP)
            z_v[j, pl.ds(q, 16)] = 1.0 / (1.0 + jnp.exp(-t))
            im = jnp.maximum(r, cc)
            jm = jnp.minimum(r, cc)
            t_v[j, pl.ds(q, 16)] = ((im * (im + 1)) >> 1) + jm
        return carry

    lax.fori_loop(0, RPT, chunk_a, 0)

    def chunk_b(j, carry):
        pltpu.async_copy(wt_hbm.at[t_v.at[j]], w_v.at[j], sem).wait()
        return carry

    lax.fori_loop(0, RPT, chunk_b, 0)

    def chunk_c(j, carry):
        for g in range(8):
            q = g * 16
            r = rows_v[j, pl.ds(q, 16)]
            cc = cols_v[j, pl.ds(q, 16)]
            ew = z_v[j, pl.ds(q, 16)] * w_v[j, pl.ds(q, 16)]
            selfm = r == cc
            ewm = jnp.where(selfm, 0.0, ew)
            ewm_v[j, pl.ds(q, 16)] = ewm
            ridx_c[pl.ds(q, 16)] = r
            dval_c[pl.ds(q, 16)] = jnp.abs(ewm)
            lval_c[pl.ds(q, 16)] = jnp.where(selfm, ew - 1.0, 0.0)
        pltpu.sync_copy(dval_c, deg_sp.at[ridx_c], add=True)
        pltpu.sync_copy(lval_c, loopd_sp.at[ridx_c], add=True)
        return carry

    lax.fori_loop(0, RPT, chunk_c, 0)
    pltpu.sync_copy(ewm_v, ewm_hbm.at[pl.ds(base, RPT)])
    plsc.subcore_barrier()

    @pl.when(s == 0)
    def _():
        pltpu.sync_copy(deg_sp, degp_hbm.at[c])
        pltpu.sync_copy(loopd_sp, loopdp_hbm.at[c])


def _k2_body(x_ref, lw_ref, dp_ref, ldp_ref, xw_ref, xs_ref, dis_ref, gl_ref):
    xw = lax.dot_general(x_ref[...], lw_ref[...], (((1,), (1,)), ((), ())),
                         preferred_element_type=jnp.float32)
    xw_ref[...] = xw
    ld = ldp_ref[...]
    dp = dp_ref[...]
    loopw = 1.0 + ld[0] + ld[1]
    deg = dp[0] + dp[1] + jnp.abs(loopw)
    dis = jnp.where(deg > 0, lax.rsqrt(jnp.where(deg > 0, deg, 1.0)), 0.0)
    xs_ref[...] = xw * dis[:, None]
    dis_ref[...] = dis[None, None]
    gl_ref[...] = (dis * dis * loopw)[None, None]


SCR = E // CHUNK // NC
TROWS = SCR // NS
SUB = 8
LSLOT = 48
CCH = 64
RROWS = CCH // 16
ROWN = N // NS
ZW = 512

_LANE16 = tuple(range(16))


def _k3_body(rows_hbm, cols_hbm, ewm_hbm, xs_hbm,
             accp_hbm,
             rows_v, cols_v, ewm_v, posb, recb, wb, gidx2, recch2, wch2,
             zli, zlf, cntl, cntb, gbuf2, acc_v, rec_sp, ewm_sp, cnt_sp,
             sem0, sem1):
    c = lax.axis_index("c")
    s = lax.axis_index("s")
    tilebase = (c * NS + s) * TROWS

    z16f = jnp.zeros((16,), jnp.float32)
    z16i = jnp.zeros((16,), jnp.int32)

    def zz(i, carry):
        zli[pl.ds(i * 16, 16)] = z16i
        zlf[pl.ds(i * 16, 16)] = z16f
        return carry

    lax.fori_loop(0, ZW // 16, zz, 0)

    def za(i, carry):
        for v in range(16):
            acc_v[i, pl.ds(v * 16, 16)] = z16f
        return carry

    lax.fori_loop(0, ROWN, za, 0)

    for i in range(NS * LSLOT * 16 // ZW):
        pltpu.sync_copy(zli,
                        rec_sp.at[pl.ds(s * NS * LSLOT * 16 + i * ZW, ZW)])
        pltpu.sync_copy(zlf,
                        ewm_sp.at[pl.ds(s * NS * LSLOT * 16 + i * ZW, ZW)])
    plsc.subcore_barrier()

    lane_iota = lax.iota(jnp.int32, 16)

    def super_p(sc, cnts):
        pltpu.sync_copy(rows_hbm.at[pl.ds(tilebase + sc * SUB, SUB)], rows_v)
        pltpu.sync_copy(cols_hbm.at[pl.ds(tilebase + sc * SUB, SUB)], cols_v)
        pltpu.sync_copy(ewm_hbm.at[pl.ds(tilebase + sc * SUB, SUB)], ewm_v)

        def produce(j, cnts2):
            cnts2 = list(cnts2)
            for g in range(8):
                q = g * 16
                r = rows_v[j, pl.ds(q, 16)]
                cc = cols_v[j, pl.ds(q, 16)]
                owner = cc >> 8
                poscnt = jnp.zeros((16,), jnp.int32)
                for o in range(16):
                    mask_o = owner == o
                    poscnt = jnp.where(mask_o, cnts2[o], poscnt)
                    cnts2[o] = cnts2[o] + jnp.where(mask_o, 1, 0)
                pos = (((s * NS + owner) * LSLOT + poscnt) * 16) + lane_iota
                posb[pl.ds(q, 16)] = pos
                recb[pl.ds(q, 16)] = r + ((cc & 255) << 12)
                wb[pl.ds(q, 16)] = ewm_v[j, pl.ds(q, 16)]
            pltpu.sync_copy(recb, rec_sp.at[posb], add=True)
            pltpu.sync_copy(wb, ewm_sp.at[posb], add=True)
            return tuple(cnts2)

        return lax.fori_loop(0, SUB, produce, cnts)

    cnts = lax.fori_loop(0, TROWS // SUB, super_p,
                         tuple(jnp.zeros((16,), jnp.int32)
                               for _ in range(16)))
    for o in range(16):
        cntb[pl.ds(o * 16, 16)] = cnts[o]
    pltpu.sync_copy(cntb, cnt_sp.at[pl.ds(s * NS * 16, NS * 16)])
    plsc.subcore_barrier()

    chs = [jnp.int32(0)]
    for p in range(NS):
        pltpu.sync_copy(cnt_sp.at[pl.ds(p * NS * 16 + s * 16, 16)], cntl)
        cntv = cntl[pl.ds(0, 16)]
        kmax = cntv[0]
        for lane in range(1, 16):
            kmax = jnp.maximum(kmax, cntv[lane])
        chs.append(chs[-1] + (kmax + (RROWS - 1)) // RROWS)
    total = chs[NS]

    def _locate(m):
        p = jnp.int32(0)
        st = jnp.int32(0)
        for i in range(NS):
            fin = chs[i + 1] <= m
            p = p + jnp.where(fin, 1, 0)
            st = jnp.where(fin, chs[i + 1], st)
        return (p * NS + s) * LSLOT * 16 + (m - st) * CCH

    def _fetch(m, par, sem):
        rbase = _locate(m)
        pltpu.sync_copy(rec_sp.at[pl.ds(rbase, CCH)], recch2.at[par])
        pltpu.sync_copy(ewm_sp.at[pl.ds(rbase, CCH)], wch2.at[par])
        for g2 in range(CCH // 16):
            q = g2 * 16
            rr = recch2[par, pl.ds(q, 16)] & 4095
            spread = (rbase + q + lane_iota) & 4095
            gidx2[par, pl.ds(q, 16)] = jnp.where(
                wch2[par, pl.ds(q, 16)] == 0.0, spread, rr)
        pltpu.async_copy(xs_hbm.at[gidx2.at[par]], gbuf2.at[par], sem)

    @pl.when(total > 0)
    def _():
        _fetch(0, 0, sem0)

    def consume(m, carry):
        par = m & 1

        @pl.when(par == 0)
        def _():
            pltpu.make_async_copy(xs_hbm.at[pl.ds(0, CCH)], gbuf2.at[0],
                                  sem0).wait()

            @pl.when(m + 1 < total)
            def _():
                _fetch(m + 1, 1, sem1)

        @pl.when(par == 1)
        def _():
            pltpu.make_async_copy(xs_hbm.at[pl.ds(0, CCH)], gbuf2.at[1],
                                  sem1).wait()

            @pl.when(m + 1 < total)
            def _():
                _fetch(m + 1, 0, sem0)

        def accum(g2, carry2):
            q = g2 * 16
            rvec = (recch2[par, pl.ds(q, 16)] >> 12) & 255
            wvec = wch2[par, pl.ds(q, 16)]
            for lane in _LANE16:
                e = q + lane
                arow = rvec[lane]
                wv = wvec[lane]
                for v in range(16):
                    plsc.addupdate(acc_v.at[arow, pl.ds(v * 16, 16)],
                                   gbuf2[par, e, pl.ds(v * 16, 16)] * wv)
            return carry2

        lax.fori_loop(0, CCH // 16, accum, 0, unroll=2)
        return carry

    lax.fori_loop(0, total, consume, 0)

    pltpu.sync_copy(acc_v, accp_hbm.at[c, pl.ds(s * ROWN, ROWN)])


def _k4_body(xw_ref, acc_ref, dis_ref, gl_ref, lb_ref, o_ref):
    i = pl.program_id(0)
    xw = xw_ref[...]
    a = acc_ref[...]
    d = dis_ref[...][0, 0]
    g = gl_ref[...][0, 0]
    agg = d[:, None] * (a[0] + a[1]) + g[:, None] * xw
    h = ALPHA * xw + (1.0 - ALPHA) * agg + lb_ref[...]
    h = jnp.maximum(h, 0.0)
    flat = ((lax.broadcasted_iota(jnp.int32, (NB, D_OUT), 0) + i * NB)
            * D_OUT + lax.broadcasted_iota(jnp.int32, (NB, D_OUT), 1))
    bits = _threefry_xor(44, lax.bitcast_convert_type(flat, jnp.uint32))
    keep = _bits_to_unif(bits) < 0.5
    o_ref[...] = jnp.where(keep, h + h, 0.0)


_SC_MESH = plsc.VectorSubcoreMesh(core_axis_name="c", subcore_axis_name="s")

_k1 = functools.partial(
    pl.kernel,
    out_type=[
        jax.ShapeDtypeStruct((E // CHUNK, CHUNK), jnp.float32),
        jax.ShapeDtypeStruct((NC, N), jnp.float32),
        jax.ShapeDtypeStruct((NC, N), jnp.float32),
    ],
    mesh=_SC_MESH,
    scratch_types=[
        pltpu.VMEM((RPT, CHUNK), jnp.int32),
        pltpu.VMEM((RPT, CHUNK), jnp.int32),
        pltpu.VMEM((RPT, CHUNK), jnp.float32),
        pltpu.VMEM((RPT, CHUNK), jnp.int32),
        pltpu.VMEM((RPT, CHUNK), jnp.float32),
        pltpu.VMEM((RPT, CHUNK), jnp.float32),
        pltpu.VMEM((CHUNK,), jnp.int32),
        pltpu.VMEM((CHUNK,), jnp.float32),
        pltpu.VMEM((CHUNK,), jnp.float32),
        pltpu.VMEM((N,), jnp.float32),
        pltpu.VMEM((16,), jnp.float32),
        pltpu.VMEM_SHARED((N,), jnp.float32),
        pltpu.VMEM_SHARED((N,), jnp.float32),
        pltpu.SemaphoreType.DMA,
    ],
)(_k1_body)

_k3 = functools.partial(
    pl.kernel,
    out_type=[
        jax.ShapeDtypeStruct((NC, N, D_OUT), jnp.float32),
    ],
    mesh=_SC_MESH,
    scratch_types=[
        pltpu.VMEM((SUB, CHUNK), jnp.int32),
        pltpu.VMEM((SUB, CHUNK), jnp.int32),
        pltpu.VMEM((SUB, CHUNK), jnp.float32),
        pltpu.VMEM((CHUNK,), jnp.int32),
        pltpu.VMEM((CHUNK,), jnp.int32),
        pltpu.VMEM((CHUNK,), jnp.float32),
        pltpu.VMEM((2, CCH), jnp.int32),
        pltpu.VMEM((2, CCH), jnp.int32),
        pltpu.VMEM((2, CCH), jnp.float32),
        pltpu.VMEM((ZW,), jnp.int32),
        pltpu.VMEM((ZW,), jnp.float32),
        pltpu.VMEM((16,), jnp.int32),
        pltpu.VMEM((NS * 16,), jnp.int32),
        pltpu.VMEM((2, CCH, D_OUT), jnp.float32),
        pltpu.VMEM((ROWN, D_OUT), jnp.float32),
        pltpu.VMEM_SHARED((NS * NS * LSLOT * 16,), jnp.int32),
        pltpu.VMEM_SHARED((NS * NS * LSLOT * 16,), jnp.float32),
        pltpu.VMEM_SHARED((NS * NS * 16,), jnp.int32),
        pltpu.SemaphoreType.DMA,
        pltpu.SemaphoreType.DMA,
    ],
)(_k3_body)


def kernel(x, edge_index, y, w_tril, a_uc, b_uc, lin_weight, lin_bias, bias):
    del y
    rows2 = edge_index[0].reshape(E // CHUNK, CHUNK)
    cols2 = edge_index[1].reshape(E // CHUNK, CHUNK)

    a = jax.nn.softplus(jnp.clip(a_uc, -10.0, None))
    b = jax.nn.softplus(jnp.clip(b_uc, -10.0, 50.0))
    u = jnp.clip(jax.random.uniform(jax.random.key(42), (1,)),
                 1e-6, 1.0 - 1e-6)
    pi = (1.0 - u ** (1.0 / b)) ** (1.0 / a)
    logits = jnp.log(pi) - jnp.log1p(-pi)
    lg16 = jnp.broadcast_to(logits, (16,)).astype(jnp.float32)

    ewm2, degp, loopdp = _k1(rows2, cols2, lg16, w_tril)

    xw, xs, dis3, gl3 = pl.pallas_call(
        _k2_body,
        grid=(N // NB,),
        in_specs=[
            pl.BlockSpec((NB, D_IN), lambda i: (i, 0)),
            pl.BlockSpec((D_OUT, D_IN), lambda i: (0, 0)),
            pl.BlockSpec((NC, NB), lambda i: (0, i)),
            pl.BlockSpec((NC, NB), lambda i: (0, i)),
        ],
        out_specs=[
            pl.BlockSpec((NB, D_OUT), lambda i: (i, 0)),
            pl.BlockSpec((NB, D_OUT), lambda i: (i, 0)),
            pl.BlockSpec((1, 1, NB), lambda i: (i, 0, 0)),
            pl.BlockSpec((1, 1, NB), lambda i: (i, 0, 0)),
        ],
        out_shape=[
            jax.ShapeDtypeStruct((N, D_OUT), jnp.float32),
            jax.ShapeDtypeStruct((N, D_OUT), jnp.float32),
            jax.ShapeDtypeStruct((N // NB, 1, NB), jnp.float32),
            jax.ShapeDtypeStruct((N // NB, 1, NB), jnp.float32),
        ],
    )(x, lin_weight, degp, loopdp)

    (accp,) = _k3(rows2, cols2, ewm2, xs)

    lb = (lin_bias + bias).reshape(1, D_OUT)
    out = pl.pallas_call(
        _k4_body,
        grid=(N // NB,),
        in_specs=[
            pl.BlockSpec((NB, D_OUT), lambda i: (i, 0)),
            pl.BlockSpec((NC, NB, D_OUT), lambda i: (0, i, 0)),
            pl.BlockSpec((1, 1, NB), lambda i: (i, 0, 0)),
            pl.BlockSpec((1, 1, NB), lambda i: (i, 0, 0)),
            pl.BlockSpec((1, D_OUT), lambda i: (0, 0)),
        ],
        out_specs=pl.BlockSpec((NB, D_OUT), lambda i: (i, 0)),
        out_shape=jax.ShapeDtypeStruct((N, D_OUT), jnp.float32),
    )(xw, accp, dis3, gl3, lb)
    return out


# speedup vs baseline: 1.3490x; 1.3490x over previous
import functools

import jax
import jax.numpy as jnp
from jax import lax
from jax.experimental import pallas as pl
from jax.experimental.pallas import tpu as pltpu
from jax.experimental.pallas import tpu_sc as plsc

N = 4096
E = 131072
D_IN = 512
D_OUT = 256
ALPHA = 0.1
TEMP = 0.67

NC = 2
NS = 16
NW = NC * NS
EPT = E // NW
CHUNK = 128
RPT = EPT // CHUNK
NB = 128


def _threefry_xor(seed_lo, lo):
    ks0 = jnp.uint32(0)
    ks1 = jnp.uint32(seed_lo)
    ks2 = jnp.uint32(0x1BD11BDA) ^ ks1
    x0 = jnp.zeros_like(lo)
    x1 = lo + ks1
    rots = ((13, 15, 26, 6), (17, 29, 16, 24))
    sched = ((ks1, ks2, 1), (ks2, ks0, 2), (ks0, ks1, 3),
             (ks1, ks2, 4), (ks2, ks0, 5))
    for g in range(5):
        for r in rots[g % 2]:
            x0 = x0 + x1
            x1 = (x1 << r) | (x1 >> (32 - r))
            x1 = x1 ^ x0
        a, b, c = sched[g]
        x0 = x0 + a
        x1 = x1 + b + jnp.uint32(c)
    return x0 ^ x1


def _bits_to_unif(bits):
    f = lax.bitcast_convert_type((bits >> 9) | jnp.uint32(0x3F800000),
                                 jnp.float32)
    return jnp.maximum(f - 1.0, 0.0)


_LN2 = 0.6931471805599453


def _slog(v):
    bits = lax.bitcast_convert_type(v, jnp.int32)
    e = (bits >> 23) - 127
    m = lax.bitcast_convert_type((bits & 0x007FFFFF) | 0x3F800000,
                                 jnp.float32)
    s = (m - 1.0) / (m + 1.0)
    s2 = s * s
    p = jnp.full_like(v, 1.0 / 11.0)
    for c in (1.0 / 9.0, 1.0 / 7.0, 1.0 / 5.0, 1.0 / 3.0, 1.0):
        p = p * s2 + jnp.float32(c)
    return e.astype(jnp.float32) * jnp.float32(_LN2) + (s + s) * p


def _k1_body(rows_hbm, cols_hbm, lg_hbm, wt_hbm,
             ewm_hbm, degp_hbm, loopdp_hbm,
             rows_v, cols_v, z_v, t_v, w_v, ewm_v, ridx_c, dval_c, lval_c,
             zero_v, lg_v, deg_sp, loopd_sp, sem):
    c = lax.axis_index("c")
    s = lax.axis_index("s")
    wid = s * NC + c
    base = wid * RPT

    pltpu.sync_copy(rows_hbm.at[pl.ds(base, RPT)], rows_v)
    pltpu.sync_copy(cols_hbm.at[pl.ds(base, RPT)], cols_v)
    pltpu.sync_copy(lg_hbm, lg_v)
    lg = lg_v[...]

    @pl.when(s == 0)
    def _():
        z16 = jnp.zeros((16,), jnp.float32)

        def zb(i, carry):
            zero_v[pl.ds(i * 16, 16)] = z16
            return carry

        lax.fori_loop(0, N // 16, zb, 0)
        pltpu.sync_copy(zero_v, deg_sp)
        pltpu.sync_copy(zero_v, loopd_sp)

    plsc.subcore_barrier()

    def chunk_a(j, carry):
        for g in range(8):
            q = g * 16
            r = rows_v[j, pl.ds(q, 16)]
            cc = cols_v[j, pl.ds(q, 16)]
            k = lax.bitcast_convert_type((r << 12) + cc, jnp.uint32)
            u2 = jnp.clip(_bits_to_unif(_threefry_xor(43, k)),
                          1e-6, 1.0 - 1e-6)
            cst = _slog(u2 / (1.0 - u2))
            t = (lg + cst) * jnp.float32(1.0 / TEMP)
            z_v[j, pl.ds(q, 16)] = 1.0 / (1.0 + jnp.exp(-t))
            im = jnp.maximum(r, cc)
            jm = jnp.minimum(r, cc)
            t_v[j, pl.ds(q, 16)] = ((im * (im + 1)) >> 1) + jm
        return carry

    lax.fori_loop(0, RPT, chunk_a, 0)

    def chunk_b(j, carry):
        pltpu.async_copy(wt_hbm.at[t_v.at[j]], w_v.at[j], sem)
        return carry

    lax.fori_loop(0, RPT, chunk_b, 0)

    def chunk_b_drain(j, carry):
        pltpu.make_async_copy(wt_hbm.at[pl.ds(0, CHUNK)], w_v.at[j],
                              sem).wait()
        return carry

    lax.fori_loop(0, RPT, chunk_b_drain, 0)

    def chunk_c(j, carry):
        for g in range(8):
            q = g * 16
            r = rows_v[j, pl.ds(q, 16)]
            cc = cols_v[j, pl.ds(q, 16)]
            ew = z_v[j, pl.ds(q, 16)] * w_v[j, pl.ds(q, 16)]
            selfm = r == cc
            ewm = jnp.where(selfm, 0.0, ew)
            ewm_v[j, pl.ds(q, 16)] = ewm
            ridx_c[pl.ds(q, 16)] = r
            dval_c[pl.ds(q, 16)] = jnp.abs(ewm)
            lval_c[pl.ds(q, 16)] = jnp.where(selfm, ew - 1.0, 0.0)
        pltpu.sync_copy(dval_c, deg_sp.at[ridx_c], add=True)
        pltpu.sync_copy(lval_c, loopd_sp.at[ridx_c], add=True)
        return carry

    lax.fori_loop(0, RPT, chunk_c, 0)
    pltpu.sync_copy(ewm_v, ewm_hbm.at[pl.ds(base, RPT)])
    plsc.subcore_barrier()

    @pl.when(s == 0)
    def _():
        pltpu.sync_copy(deg_sp, degp_hbm.at[c])
        pltpu.sync_copy(loopd_sp, loopdp_hbm.at[c])


def _k2_body(x_ref, lw_ref, dp_ref, ldp_ref, xw_ref, xs_ref, dis_ref, gl_ref):
    xw = lax.dot_general(x_ref[...], lw_ref[...], (((1,), (1,)), ((), ())),
                         preferred_element_type=jnp.float32)
    xw_ref[...] = xw
    ld = ldp_ref[...]
    dp = dp_ref[...]
    loopw = 1.0 + ld[0] + ld[1]
    deg = dp[0] + dp[1] + jnp.abs(loopw)
    dis = jnp.where(deg > 0, lax.rsqrt(jnp.where(deg > 0, deg, 1.0)), 0.0)
    xs_ref[...] = xw * dis[:, None]
    dis_ref[...] = dis[None, None]
    gl_ref[...] = (dis * dis * loopw)[None, None]


SCR = E // CHUNK // NC
TROWS = SCR // NS
SUB = 8
LSLOT = 48
CCH = 64
RROWS = CCH // 16
ROWN = N // NS
ZW = 512

_LANE16 = tuple(range(16))


def _k3_body(rows_hbm, cols_hbm, ewm_hbm, xs_hbm,
             accp_hbm,
             rows_v, cols_v, ewm_v, posb, recb, wb, gidx2, recch2, wch2,
             zli, zlf, cntl, cntb, gbuf2, acc_v, rec_sp, ewm_sp, cnt_sp,
             sem0, sem1):
    c = lax.axis_index("c")
    s = lax.axis_index("s")
    tilebase = (c * NS + s) * TROWS

    z16f = jnp.zeros((16,), jnp.float32)
    z16i = jnp.zeros((16,), jnp.int32)

    def zz(i, carry):
        zli[pl.ds(i * 16, 16)] = z16i
        zlf[pl.ds(i * 16, 16)] = z16f
        return carry

    lax.fori_loop(0, ZW // 16, zz, 0)

    def za(i, carry):
        for v in range(16):
            acc_v[i, pl.ds(v * 16, 16)] = z16f
        return carry

    lax.fori_loop(0, ROWN, za, 0)

    for i in range(NS * LSLOT * 16 // ZW):
        pltpu.sync_copy(zli,
                        rec_sp.at[pl.ds(s * NS * LSLOT * 16 + i * ZW, ZW)])
        pltpu.sync_copy(zlf,
                        ewm_sp.at[pl.ds(s * NS * LSLOT * 16 + i * ZW, ZW)])
    plsc.subcore_barrier()

    lane_iota = lax.iota(jnp.int32, 16)

    def super_p(sc, cnts):
        pltpu.sync_copy(rows_hbm.at[pl.ds(tilebase + sc * SUB, SUB)], rows_v)
        pltpu.sync_copy(cols_hbm.at[pl.ds(tilebase + sc * SUB, SUB)], cols_v)
        pltpu.sync_copy(ewm_hbm.at[pl.ds(tilebase + sc * SUB, SUB)], ewm_v)

        def produce(j, cnts2):
            cnts2 = list(cnts2)
            for g in range(8):
                q = g * 16
                r = rows_v[j, pl.ds(q, 16)]
                cc = cols_v[j, pl.ds(q, 16)]
                owner = cc >> 8
                poscnt = jnp.zeros((16,), jnp.int32)
                for o in range(16):
                    mask_o = owner == o
                    poscnt = jnp.where(mask_o, cnts2[o], poscnt)
                    cnts2[o] = cnts2[o] + jnp.where(mask_o, 1, 0)
                pos = (((s * NS + owner) * LSLOT + poscnt) * 16) + lane_iota
                posb[pl.ds(q, 16)] = pos
                recb[pl.ds(q, 16)] = r + ((cc & 255) << 12)
                wb[pl.ds(q, 16)] = ewm_v[j, pl.ds(q, 16)]
            pltpu.sync_copy(recb, rec_sp.at[posb], add=True)
            pltpu.sync_copy(wb, ewm_sp.at[posb], add=True)
            return tuple(cnts2)

        return lax.fori_loop(0, SUB, produce, cnts)

    cnts = lax.fori_loop(0, TROWS // SUB, super_p,
                         tuple(jnp.zeros((16,), jnp.int32)
                               for _ in range(16)))
    for o in range(16):
        cntb[pl.ds(o * 16, 16)] = cnts[o]
    pltpu.sync_copy(cntb, cnt_sp.at[pl.ds(s * NS * 16, NS * 16)])
    plsc.subcore_barrier()

    chs = [jnp.int32(0)]
    for p in range(NS):
        pltpu.sync_copy(cnt_sp.at[pl.ds(p * NS * 16 + s * 16, 16)], cntl)
        cntv = cntl[pl.ds(0, 16)]
        kmax = cntv[0]
        for lane in range(1, 16):
            kmax = jnp.maximum(kmax, cntv[lane])
        chs.append(chs[-1] + (kmax + (RROWS - 1)) // RROWS)
    total = chs[NS]

    def _locate(m):
        p = jnp.int32(0)
        st = jnp.int32(0)
        for i in range(NS):
            fin = chs[i + 1] <= m
            p = p + jnp.where(fin, 1, 0)
            st = jnp.where(fin, chs[i + 1], st)
        return (p * NS + s) * LSLOT * 16 + (m - st) * CCH

    def _fetch(m, par, sem):
        rbase = _locate(m)
        pltpu.sync_copy(rec_sp.at[pl.ds(rbase, CCH)], recch2.at[par])
        pltpu.sync_copy(ewm_sp.at[pl.ds(rbase, CCH)], wch2.at[par])
        for g2 in range(CCH // 16):
            q = g2 * 16
            rr = recch2[par, pl.ds(q, 16)] & 4095
            spread = (rbase + q + lane_iota) & 4095
            gidx2[par, pl.ds(q, 16)] = jnp.where(
                wch2[par, pl.ds(q, 16)] == 0.0, spread, rr)
        pltpu.async_copy(xs_hbm.at[gidx2.at[par]], gbuf2.at[par], sem)

    @pl.when(total > 0)
    def _():
        _fetch(0, 0, sem0)

    def consume(m, carry):
        par = m & 1

        @pl.when(par == 0)
        def _():
            pltpu.make_async_copy(xs_hbm.at[pl.ds(0, CCH)], gbuf2.at[0],
                                  sem0).wait()

            @pl.when(m + 1 < total)
            def _():
                _fetch(m + 1, 1, sem1)

        @pl.when(par == 1)
        def _():
            pltpu.make_async_copy(xs_hbm.at[pl.ds(0, CCH)], gbuf2.at[1],
                                  sem1).wait()

            @pl.when(m + 1 < total)
            def _():
                _fetch(m + 1, 0, sem0)

        def accum(g2, carry2):
            q = g2 * 16
            rvec = (recch2[par, pl.ds(q, 16)] >> 12) & 255
            wvec = wch2[par, pl.ds(q, 16)]
            for lane in _LANE16:
                e = q + lane
                arow = rvec[lane]
                wv = wvec[lane]
                for v in range(16):
                    plsc.addupdate(acc_v.at[arow, pl.ds(v * 16, 16)],
                                   gbuf2[par, e, pl.ds(v * 16, 16)] * wv)
            return carry2

        lax.fori_loop(0, CCH // 16, accum, 0)
        return carry

    lax.fori_loop(0, total, consume, 0)

    pltpu.sync_copy(acc_v, accp_hbm.at[c, pl.ds(s * ROWN, ROWN)])


def _k4_body(xw_ref, acc_ref, dis_ref, gl_ref, lb_ref, o_ref):
    i = pl.program_id(0)
    xw = xw_ref[...]
    a = acc_ref[...]
    d = dis_ref[...][0, 0]
    g = gl_ref[...][0, 0]
    agg = d[:, None] * (a[0] + a[1]) + g[:, None] * xw
    h = ALPHA * xw + (1.0 - ALPHA) * agg + lb_ref[...]
    h = jnp.maximum(h, 0.0)
    flat = ((lax.broadcasted_iota(jnp.int32, (NB, D_OUT), 0) + i * NB)
            * D_OUT + lax.broadcasted_iota(jnp.int32, (NB, D_OUT), 1))
    bits = _threefry_xor(44, lax.bitcast_convert_type(flat, jnp.uint32))
    keep = _bits_to_unif(bits) < 0.5
    o_ref[...] = jnp.where(keep, h + h, 0.0)


_SC_MESH = plsc.VectorSubcoreMesh(core_axis_name="c", subcore_axis_name="s")

_k1 = functools.partial(
    pl.kernel,
    out_type=[
        jax.ShapeDtypeStruct((E // CHUNK, CHUNK), jnp.float32),
        jax.ShapeDtypeStruct((NC, N), jnp.float32),
        jax.ShapeDtypeStruct((NC, N), jnp.float32),
    ],
    mesh=_SC_MESH,
    scratch_types=[
        pltpu.VMEM((RPT, CHUNK), jnp.int32),
        pltpu.VMEM((RPT, CHUNK), jnp.int32),
        pltpu.VMEM((RPT, CHUNK), jnp.float32),
        pltpu.VMEM((RPT, CHUNK), jnp.int32),
        pltpu.VMEM((RPT, CHUNK), jnp.float32),
        pltpu.VMEM((RPT, CHUNK), jnp.float32),
        pltpu.VMEM((CHUNK,), jnp.int32),
        pltpu.VMEM((CHUNK,), jnp.float32),
        pltpu.VMEM((CHUNK,), jnp.float32),
        pltpu.VMEM((N,), jnp.float32),
        pltpu.VMEM((16,), jnp.float32),
        pltpu.VMEM_SHARED((N,), jnp.float32),
        pltpu.VMEM_SHARED((N,), jnp.float32),
        pltpu.SemaphoreType.DMA,
    ],
)(_k1_body)

_k3 = functools.partial(
    pl.kernel,
    out_type=[
        jax.ShapeDtypeStruct((NC, N, D_OUT), jnp.float32),
    ],
    mesh=_SC_MESH,
    scratch_types=[
        pltpu.VMEM((SUB, CHUNK), jnp.int32),
        pltpu.VMEM((SUB, CHUNK), jnp.int32),
        pltpu.VMEM((SUB, CHUNK), jnp.float32),
        pltpu.VMEM((CHUNK,), jnp.int32),
        pltpu.VMEM((CHUNK,), jnp.int32),
        pltpu.VMEM((CHUNK,), jnp.float32),
        pltpu.VMEM((2, CCH), jnp.int32),
        pltpu.VMEM((2, CCH), jnp.int32),
        pltpu.VMEM((2, CCH), jnp.float32),
        pltpu.VMEM((ZW,), jnp.int32),
        pltpu.VMEM((ZW,), jnp.float32),
        pltpu.VMEM((16,), jnp.int32),
        pltpu.VMEM((NS * 16,), jnp.int32),
        pltpu.VMEM((2, CCH, D_OUT), jnp.float32),
        pltpu.VMEM((ROWN, D_OUT), jnp.float32),
        pltpu.VMEM_SHARED((NS * NS * LSLOT * 16,), jnp.int32),
        pltpu.VMEM_SHARED((NS * NS * LSLOT * 16,), jnp.float32),
        pltpu.VMEM_SHARED((NS * NS * 16,), jnp.int32),
        pltpu.SemaphoreType.DMA,
        pltpu.SemaphoreType.DMA,
    ],
)(_k3_body)


def kernel(x, edge_index, y, w_tril, a_uc, b_uc, lin_weight, lin_bias, bias):
    del y
    rows2 = edge_index[0].reshape(E // CHUNK, CHUNK)
    cols2 = edge_index[1].reshape(E // CHUNK, CHUNK)

    a = jax.nn.softplus(jnp.clip(a_uc, -10.0, None))
    b = jax.nn.softplus(jnp.clip(b_uc, -10.0, 50.0))
    u = jnp.clip(jax.random.uniform(jax.random.key(42), (1,)),
                 1e-6, 1.0 - 1e-6)
    pi = (1.0 - u ** (1.0 / b)) ** (1.0 / a)
    logits = jnp.log(pi) - jnp.log1p(-pi)
    lg16 = jnp.broadcast_to(logits, (16,)).astype(jnp.float32)

    ewm2, degp, loopdp = _k1(rows2, cols2, lg16, w_tril)

    xw, xs, dis3, gl3 = pl.pallas_call(
        _k2_body,
        grid=(N // NB,),
        in_specs=[
            pl.BlockSpec((NB, D_IN), lambda i: (i, 0)),
            pl.BlockSpec((D_OUT, D_IN), lambda i: (0, 0)),
            pl.BlockSpec((NC, NB), lambda i: (0, i)),
            pl.BlockSpec((NC, NB), lambda i: (0, i)),
        ],
        out_specs=[
            pl.BlockSpec((NB, D_OUT), lambda i: (i, 0)),
            pl.BlockSpec((NB, D_OUT), lambda i: (i, 0)),
            pl.BlockSpec((1, 1, NB), lambda i: (i, 0, 0)),
            pl.BlockSpec((1, 1, NB), lambda i: (i, 0, 0)),
        ],
        out_shape=[
            jax.ShapeDtypeStruct((N, D_OUT), jnp.float32),
            jax.ShapeDtypeStruct((N, D_OUT), jnp.float32),
            jax.ShapeDtypeStruct((N // NB, 1, NB), jnp.float32),
            jax.ShapeDtypeStruct((N // NB, 1, NB), jnp.float32),
        ],
    )(x, lin_weight, degp, loopdp)

    (accp,) = _k3(rows2, cols2, ewm2, xs)

    lb = (lin_bias + bias).reshape(1, D_OUT)
    out = pl.pallas_call(
        _k4_body,
        grid=(N // NB,),
        in_specs=[
            pl.BlockSpec((NB, D_OUT), lambda i: (i, 0)),
            pl.BlockSpec((NC, NB, D_OUT), lambda i: (0, i, 0)),
            pl.BlockSpec((1, 1, NB), lambda i: (i, 0, 0)),
            pl.BlockSpec((1, 1, NB), lambda i: (i, 0, 0)),
            pl.BlockSpec((1, D_OUT), lambda i: (0, 0)),
        ],
        out_specs=pl.BlockSpec((NB, D_OUT), lambda i: (i, 0)),
        out_shape=jax.ShapeDtypeStruct((N, D_OUT), jnp.float32),
    )(xw, accp, dis3, gl3, lb)
    return out
